# Initial kernel scaffold; baseline (speedup 1.0000x reference)
import jax
import jax.numpy as jnp
from jax.experimental import pallas as pl


def kernel(z_e_x, codebook):
    raise NotImplementedError("write your pallas kernel here")


# speedup vs baseline: 3.2837x; 3.2837x over previous
import jax
import jax.numpy as jnp
from jax.experimental import pallas as pl

_CODEBOOK_SIZE = 1024
_CODE_SIZE = 256
_BETA = 0.25
_M_BLOCK = 1024


def _vq_loss_kernel(z_ref, cb_ref, out_ref):
    z = z_ref[...]
    cb = cb_ref[...]
    neg2zc = jax.lax.dot_general(
        z, cb,
        dimension_numbers=(((1,), (1,)), ((), ())),
        preferred_element_type=jnp.float32,
    ) * -2.0
    csqr = jnp.sum(cb * cb, axis=1)
    d = neg2zc + csqr[None, :]
    m = jnp.min(d, axis=1)
    zsqr = jnp.sum(z * z, axis=1)
    out_ref[...] = (1.0 + _BETA) * (zsqr + m)


@jax.jit
def kernel(z_e_x, codebook):
    batch = z_e_x.shape[0]
    grid = (batch // _M_BLOCK,)
    return pl.pallas_call(
        _vq_loss_kernel,
        grid=grid,
        in_specs=[
            pl.BlockSpec((_M_BLOCK, _CODE_SIZE), lambda i: (i, 0)),
            pl.BlockSpec((_CODEBOOK_SIZE, _CODE_SIZE), lambda i: (0, 0)),
        ],
        out_specs=pl.BlockSpec((_M_BLOCK,), lambda i: (i,)),
        out_shape=jax.ShapeDtypeStruct((batch,), jnp.float32),
    )(z_e_x, codebook)


# speedup vs baseline: 5.9675x; 1.8173x over previous
import jax
import jax.numpy as jnp
from jax.experimental import pallas as pl

_CODEBOOK_SIZE = 1024
_CODE_SIZE = 256
_BETA = 0.25
_M_BLOCK = 1024


def _vq_loss_kernel(z_ref, cb_ref, out_ref):
    z = z_ref[...]
    cb = cb_ref[...]
    zc = jax.lax.dot_general(
        cb, z,
        dimension_numbers=(((1,), (1,)), ((), ())),
        preferred_element_type=jnp.float32,
    )
    ones_k = jnp.ones((1, _CODE_SIZE), dtype=jnp.float32)
    half_csqr = jax.lax.dot_general(
        cb * (0.5 * cb), ones_k,
        dimension_numbers=(((1,), (1,)), ((), ())),
        preferred_element_type=jnp.float32,
    )
    zsqr = jax.lax.dot_general(
        ones_k, z * z,
        dimension_numbers=(((1,), (1,)), ((), ())),
        preferred_element_type=jnp.float32,
    )
    mx = jnp.max(zc - half_csqr, axis=0)
    out_ref[...] = (1.0 + _BETA) * zsqr[0] - (2.0 + 2.0 * _BETA) * mx


@jax.jit
def kernel(z_e_x, codebook):
    batch = z_e_x.shape[0]
    grid = (batch // _M_BLOCK,)
    return pl.pallas_call(
        _vq_loss_kernel,
        grid=grid,
        in_specs=[
            pl.BlockSpec((_M_BLOCK, _CODE_SIZE), lambda i: (i, 0)),
            pl.BlockSpec((_CODEBOOK_SIZE, _CODE_SIZE), lambda i: (0, 0)),
        ],
        out_specs=pl.BlockSpec((_M_BLOCK,), lambda i: (i,)),
        out_shape=jax.ShapeDtypeStruct((batch,), jnp.float32),
    )(z_e_x, codebook)
